# Initial kernel scaffold; baseline (speedup 1.0000x reference)
import jax
import jax.numpy as jnp
from jax.experimental import pallas as pl


def kernel(adj_indices, adj_values, embeds):
    raise NotImplementedError("write your pallas kernel here")


# speedup vs baseline: 4.5395x; 4.5395x over previous
import functools
import jax
import jax.numpy as jnp
from jax import lax
from jax.experimental import pallas as pl
from jax.experimental.pallas import tpu as pltpu
from jax.experimental.pallas import tpu_sc as plsc

N = 10000
E = 320000
D = 128

NC = 2
NS = 16
NW = NC * NS
EPW = E // NW
C = 80
NCH = EPW // C
RPS = 624
TAIL = N - NS * RPS
ZR = 104
LANES = 16
DV = D // LANES


def _sc_spmm(row, col, val, embeds):
    mesh = plsc.VectorSubcoreMesh(
        core_axis_name="c", subcore_axis_name="s", num_cores=NC, num_subcores=NS
    )

    @functools.partial(
        pl.kernel,
        out_type=(
            jax.ShapeDtypeStruct((N, D), jnp.float32),
            jax.ShapeDtypeStruct((N, D), jnp.float32),
        ),
        mesh=mesh,
        scratch_types=[
            pltpu.VMEM_SHARED((N, D), jnp.float32),
            pltpu.VMEM((C,), jnp.int32),
            pltpu.VMEM((C,), jnp.int32),
            pltpu.VMEM((C,), jnp.float32),
            pltpu.VMEM((C, D), jnp.float32),
            pltpu.VMEM((ZR, D), jnp.float32),
            pltpu.SemaphoreType.DMA,
        ],
    )
    def spmm(row_hbm, col_hbm, val_hbm, emb_hbm, out0, out1,
             acc, colv, rowv, valv, rows, zbuf, sem):
        cid = lax.axis_index("c")
        sid = lax.axis_index("s")
        wid = sid * NC + cid

        zv = jnp.zeros((LANES,), jnp.float32)

        def zrow(i, carry):
            for d in range(DV):
                zbuf[i, pl.ds(d * LANES, LANES)] = zv
            return carry

        lax.fori_loop(0, ZR, zrow, 0)
        for k in range(RPS // ZR):
            pltpu.sync_copy(zbuf, acc.at[pl.ds(sid * RPS + k * ZR, ZR)])

        @pl.when(sid == NS - 1)
        def _():
            pltpu.sync_copy(zbuf.at[pl.ds(0, TAIL)], acc.at[pl.ds(NS * RPS, TAIL)])

        plsc.subcore_barrier()

        def chunk(j, carry):
            base = wid * EPW + j * C
            pltpu.sync_copy(col_hbm.at[pl.ds(base, C)], colv)
            pltpu.sync_copy(row_hbm.at[pl.ds(base, C)], rowv)
            pltpu.sync_copy(val_hbm.at[pl.ds(base, C)], valv)
            pltpu.async_copy(emb_hbm.at[colv], rows, sem).wait()

            def scale16(g, c2):
                vals16 = valv[pl.ds(g * LANES, LANES)]
                for i in range(LANES):
                    e = g * LANES + i
                    s = vals16.at[jnp.full((LANES,), i, jnp.int32)].get(
                        mode="promise_in_bounds")
                    for d in range(DV):
                        sl = pl.ds(d * LANES, LANES)
                        rows[e, sl] = rows[e, sl] * s
                return c2

            lax.fori_loop(0, C // LANES, scale16, 0)
            pltpu.sync_copy(rows, acc.at[rowv], add=True)
            return carry

        lax.fori_loop(0, NCH, chunk, 0)
        plsc.subcore_barrier()

        @pl.when(cid == 0)
        def _():
            pltpu.sync_copy(acc.at[pl.ds(sid * RPS, RPS)],
                            out0.at[pl.ds(sid * RPS, RPS)])

            @pl.when(sid == NS - 1)
            def _():
                pltpu.sync_copy(acc.at[pl.ds(NS * RPS, TAIL)],
                                out0.at[pl.ds(NS * RPS, TAIL)])

        @pl.when(cid == 1)
        def _():
            pltpu.sync_copy(acc.at[pl.ds(sid * RPS, RPS)],
                            out1.at[pl.ds(sid * RPS, RPS)])

            @pl.when(sid == NS - 1)
            def _():
                pltpu.sync_copy(acc.at[pl.ds(NS * RPS, TAIL)],
                                out1.at[pl.ds(NS * RPS, TAIL)])

    return spmm(row, col, val, embeds)


def _merge_body(a_ref, b_ref, o_ref):
    o_ref[...] = a_ref[...] + b_ref[...]


def _merge(a, b):
    blk = 1000
    return pl.pallas_call(
        _merge_body,
        out_shape=jax.ShapeDtypeStruct((N, D), jnp.float32),
        grid=(N // blk,),
        in_specs=[
            pl.BlockSpec((blk, D), lambda i: (i, 0)),
            pl.BlockSpec((blk, D), lambda i: (i, 0)),
        ],
        out_specs=pl.BlockSpec((blk, D), lambda i: (i, 0)),
    )(a, b)


def kernel(adj_indices, adj_values, embeds):
    row = adj_indices[0].astype(jnp.int32)
    col = adj_indices[1].astype(jnp.int32)
    out0, out1 = _sc_spmm(row, col, adj_values, embeds)
    return _merge(out0, out1)
